# baseline (speedup 1.0000x reference)
import functools

import jax
import jax.numpy as jnp
from jax import lax
from jax.experimental import pallas as pl
from jax.experimental.pallas import tpu as pltpu
from jax.experimental.pallas import tpu_sc as plsc

_M, _P, _DIM = 2048, 128, 128
_NC, _NS = 2, 16
_NW = _NC * _NS
_BW = _M // _NW
_BM = 64


def _emb_lookup_sc(tbl, tcd):
    mesh = plsc.VectorSubcoreMesh(core_axis_name="c", subcore_axis_name="s")

    @functools.partial(
        pl.kernel,
        mesh=mesh,
        out_type=jax.ShapeDtypeStruct((_M, _DIM), jnp.float32),
        scratch_types=[
            pltpu.VMEM((3 * _BW,), jnp.int32),
            pltpu.VMEM((_BW,), jnp.int32),
            pltpu.VMEM((_BW, _DIM), jnp.float32),
            pltpu.SemaphoreType.DMA,
        ],
    )
    def k(tbl_hbm, tcd_hbm, out_hbm, tcd_v, idx_v, rows_v, sem):
        wid = lax.axis_index("s") * _NC + lax.axis_index("c")
        base = wid * _BW
        pltpu.sync_copy(tcd_hbm.at[pl.ds(base, _BW)], tcd_v.at[pl.ds(0, _BW)])
        pltpu.sync_copy(tcd_hbm.at[pl.ds(_M + base, _BW)],
                        tcd_v.at[pl.ds(_BW, _BW)])
        pltpu.sync_copy(tcd_hbm.at[pl.ds(2 * _M + base, _BW)],
                        tcd_v.at[pl.ds(2 * _BW, _BW)])
        for j in range(_BW // 16):
            idx_v[pl.ds(j * 16, 16)] = (
                tcd_v[pl.ds(j * 16, 16)] * 6
                + tcd_v[pl.ds(_BW + j * 16, 16)] * 3
                + tcd_v[pl.ds(2 * _BW + j * 16, 16)])
        pltpu.async_copy(tbl_hbm.at[idx_v], rows_v, sem).wait()
        pltpu.sync_copy(rows_v, out_hbm.at[pl.ds(base, _BW)])

    return k(tbl, tcd)


def _mlp_body(pv_ref, ct_ref, mk_ref, w1_ref, b1_ref, w2_ref, b2_ref,
              w3a_ref, w3b_ref, b3_ref, w4_ref, b4_ref, out_ref):
    f32, bf16 = jnp.float32, jnp.bfloat16
    x = pv_ref[...].reshape(_BM * _P, 6)
    t1 = jnp.dot(x, w1_ref[...], preferred_element_type=f32).astype(bf16)
    c1 = jnp.dot(ct_ref[...], w1_ref[0:3, :],
                 preferred_element_type=f32).astype(bf16)
    h1 = t1.reshape(_BM, _P, 128) - c1[:, None, :] + b1_ref[...]
    h1 = jnp.maximum(h1, jnp.zeros((), bf16))
    h = jnp.dot(h1.reshape(_BM * _P, 128), w2_ref[...],
                preferred_element_type=f32).astype(bf16) + b2_ref[...]
    m = mk_ref[...]
    hm = h.reshape(_BM, _P, 256) * m[:, :, None]
    pooled = jnp.max(hm, axis=1)
    t3 = jnp.dot(hm.reshape(_BM * _P, 256), w3a_ref[...],
                 preferred_element_type=f32).astype(bf16)
    p3 = jnp.dot(pooled, w3b_ref[...],
                 preferred_element_type=f32).astype(bf16) + b3_ref[...]
    g1 = jnp.maximum(t3.reshape(_BM, _P, 256) + p3[:, None, :],
                     jnp.zeros((), bf16))
    g = jnp.dot(g1.reshape(_BM * _P, 256), w4_ref[...],
                preferred_element_type=f32).astype(bf16) + b4_ref[...]
    gm = g.reshape(_BM, _P, _DIM) * m[:, :, None]
    out_ref[...] = jnp.max(gm, axis=1).astype(f32)


def _mlp_tc(posvec, center, maskf, w1f, b1f, w2, b2, w3a, w3b, b3f, w4, b4):
    const = lambda *dims: pl.BlockSpec(dims, lambda i: (0,) * len(dims))
    return pl.pallas_call(
        _mlp_body,
        grid=(_M // _BM,),
        in_specs=[
            pl.BlockSpec((_BM, _P, 6), lambda i: (i, 0, 0)),
            pl.BlockSpec((_BM, 3), lambda i: (i, 0)),
            pl.BlockSpec((_BM, _P), lambda i: (i, 0)),
            const(6, 128), const(1, 128),
            const(128, 256), const(1, 256),
            const(256, 256), const(256, 256), const(1, 256),
            const(256, _DIM), const(1, _DIM),
        ],
        out_specs=pl.BlockSpec((_BM, _DIM), lambda i: (i, 0)),
        out_shape=jax.ShapeDtypeStruct((_M, _DIM), jnp.float32),
        compiler_params=pltpu.CompilerParams(
            dimension_semantics=("parallel",)),
    )(posvec, center, maskf, w1f, b1f, w2, b2, w3a, w3b, b3f, w4, b4)


def kernel(q_lane_type, q_point_position, q_point_vector, q_lane_control,
           q_lane_direction, q_lane_center, q_valid_mask,
           w1, b1, bn1_g, bn1_b, w2, b2, w3, b3, bn2_g, bn2_b, w4, b4,
           type_emb, control_emb, direction_emb):
    bf16 = jnp.bfloat16
    posvec = jnp.concatenate(
        [q_point_position, q_point_vector], axis=-1).astype(bf16)
    maskf = q_valid_mask.astype(bf16)
    w1f = w1 * bn1_g[None, :]
    b1f = (b1 * bn1_g + bn1_b)[None, :]
    w3f = w3 * bn2_g[None, :]
    b3f = ((b3 * bn2_g) + bn2_b)[None, :]
    w3a, w3b = w3f[:256], w3f[256:]
    tbl = (type_emb[:, None, None, :] + control_emb[None, :, None, :]
           + direction_emb[None, None, :, :]).reshape(12, _DIM)
    tcd = jnp.concatenate([q_lane_type.astype(jnp.int32),
                           q_lane_control.astype(jnp.int32),
                           q_lane_direction.astype(jnp.int32)])
    emb = _emb_lookup_sc(tbl, tcd)
    x = _mlp_tc(posvec, q_lane_center.astype(bf16), maskf,
                w1f.astype(bf16), b1f.astype(bf16),
                w2.astype(bf16), b2[None, :].astype(bf16),
                w3a.astype(bf16), w3b.astype(bf16), b3f.astype(bf16),
                w4.astype(bf16), b4[None, :].astype(bf16))
    return ((x + emb)[None], q_valid_mask[None])


# speedup vs baseline: 1.0379x; 1.0379x over previous
import functools

import jax
import jax.numpy as jnp
from jax import lax
from jax.experimental import pallas as pl
from jax.experimental.pallas import tpu as pltpu
from jax.experimental.pallas import tpu_sc as plsc

_M, _P, _DIM = 2048, 128, 128
_NC, _NS = 2, 16
_NW = _NC * _NS
_BW = _M // _NW
_BM = 128


def _emb_lookup_sc(tbl, tcd):
    mesh = plsc.VectorSubcoreMesh(core_axis_name="c", subcore_axis_name="s")

    @functools.partial(
        pl.kernel,
        mesh=mesh,
        out_type=jax.ShapeDtypeStruct((_M, _DIM), jnp.float32),
        scratch_types=[
            pltpu.VMEM((3 * _BW,), jnp.int32),
            pltpu.VMEM((_BW,), jnp.int32),
            pltpu.VMEM((_BW, _DIM), jnp.float32),
            pltpu.SemaphoreType.DMA,
            pltpu.SemaphoreType.DMA,
            pltpu.SemaphoreType.DMA,
        ],
    )
    def k(tbl_hbm, tcd_hbm, out_hbm, tcd_v, idx_v, rows_v, sem, sem2, sem3):
        wid = lax.axis_index("s") * _NC + lax.axis_index("c")
        base = wid * _BW
        cp1 = pltpu.async_copy(tcd_hbm.at[pl.ds(base, _BW)],
                               tcd_v.at[pl.ds(0, _BW)], sem)
        cp2 = pltpu.async_copy(tcd_hbm.at[pl.ds(_M + base, _BW)],
                               tcd_v.at[pl.ds(_BW, _BW)], sem2)
        cp3 = pltpu.async_copy(tcd_hbm.at[pl.ds(2 * _M + base, _BW)],
                               tcd_v.at[pl.ds(2 * _BW, _BW)], sem3)
        cp1.wait(); cp2.wait(); cp3.wait()
        for j in range(_BW // 16):
            idx_v[pl.ds(j * 16, 16)] = (
                tcd_v[pl.ds(j * 16, 16)] * 6
                + tcd_v[pl.ds(_BW + j * 16, 16)] * 3
                + tcd_v[pl.ds(2 * _BW + j * 16, 16)])
        pltpu.async_copy(tbl_hbm.at[idx_v], rows_v, sem).wait()
        pltpu.sync_copy(rows_v, out_hbm.at[pl.ds(base, _BW)])

    return k(tbl, tcd)


def _mlp_body(pv_ref, ct_ref, mk_ref, emb_ref, w1_ref, b1_ref, w2_ref, b2_ref,
              w3a_ref, w3b_ref, b3_ref, w4_ref, b4_ref, out_ref):
    f32, bf16 = jnp.float32, jnp.bfloat16
    x = pv_ref[...].reshape(_BM * _P, 6)
    t1 = jnp.dot(x, w1_ref[...], preferred_element_type=f32).astype(bf16)
    c1 = b1_ref[...] - jnp.dot(ct_ref[...], w1_ref[0:3, :],
                               preferred_element_type=f32).astype(bf16)
    h1 = t1.reshape(_BM, _P, 128) + c1[:, None, :]
    h1 = jnp.maximum(h1, jnp.zeros((), bf16))
    h = jnp.dot(h1.reshape(_BM * _P, 128), w2_ref[...],
                preferred_element_type=f32).astype(bf16) + b2_ref[...]
    m = mk_ref[...]
    hm = h.reshape(_BM, _P, 256) * m[:, :, None]
    pooled = jnp.max(hm, axis=1)
    t3 = jnp.dot(hm.reshape(_BM * _P, 256), w3a_ref[...],
                 preferred_element_type=f32).astype(bf16)
    p3 = jnp.dot(pooled, w3b_ref[...],
                 preferred_element_type=f32).astype(bf16) + b3_ref[...]
    g1 = jnp.maximum(t3.reshape(_BM, _P, 256) + p3[:, None, :],
                     jnp.zeros((), bf16))
    g = jnp.dot(g1.reshape(_BM * _P, 256), w4_ref[...],
                preferred_element_type=f32).astype(bf16) + b4_ref[...]
    gm = g.reshape(_BM, _P, _DIM) * m[:, :, None]
    out_ref[...] = jnp.max(gm, axis=1).astype(f32) + emb_ref[...]


def _mlp_tc(posvec, center, maskf, emb, w1f, b1f, w2, b2, w3a, w3b, b3f, w4, b4):
    const = lambda *dims: pl.BlockSpec(dims, lambda i: (0,) * len(dims))
    return pl.pallas_call(
        _mlp_body,
        grid=(_M // _BM,),
        in_specs=[
            pl.BlockSpec((_BM, _P, 6), lambda i: (i, 0, 0)),
            pl.BlockSpec((_BM, 3), lambda i: (i, 0)),
            pl.BlockSpec((_BM, _P), lambda i: (i, 0)),
            pl.BlockSpec((_BM, _DIM), lambda i: (i, 0)),
            const(6, 128), const(1, 128),
            const(128, 256), const(1, 256),
            const(256, 256), const(256, 256), const(1, 256),
            const(256, _DIM), const(1, _DIM),
        ],
        out_specs=pl.BlockSpec((_BM, _DIM), lambda i: (i, 0)),
        out_shape=jax.ShapeDtypeStruct((_M, _DIM), jnp.float32),
        compiler_params=pltpu.CompilerParams(
            dimension_semantics=("parallel",)),
    )(posvec, center, maskf, emb, w1f, b1f, w2, b2, w3a, w3b, b3f, w4, b4)


def kernel(q_lane_type, q_point_position, q_point_vector, q_lane_control,
           q_lane_direction, q_lane_center, q_valid_mask,
           w1, b1, bn1_g, bn1_b, w2, b2, w3, b3, bn2_g, bn2_b, w4, b4,
           type_emb, control_emb, direction_emb):
    bf16 = jnp.bfloat16
    posvec = jnp.concatenate(
        [q_point_position, q_point_vector], axis=-1).astype(bf16)
    maskf = q_valid_mask.astype(bf16)
    w1f = w1 * bn1_g[None, :]
    b1f = (b1 * bn1_g + bn1_b)[None, :]
    w3f = w3 * bn2_g[None, :]
    b3f = ((b3 * bn2_g) + bn2_b)[None, :]
    w3a, w3b = w3f[:256], w3f[256:]
    tbl = (type_emb[:, None, None, :] + control_emb[None, :, None, :]
           + direction_emb[None, None, :, :]).reshape(12, _DIM)
    tcd = jnp.concatenate([q_lane_type.astype(jnp.int32),
                           q_lane_control.astype(jnp.int32),
                           q_lane_direction.astype(jnp.int32)])
    emb = _emb_lookup_sc(tbl, tcd)
    x = _mlp_tc(posvec, q_lane_center.astype(bf16), maskf, emb,
                w1f.astype(bf16), b1f.astype(bf16),
                w2.astype(bf16), b2[None, :].astype(bf16),
                w3a.astype(bf16), w3b.astype(bf16), b3f.astype(bf16),
                w4.astype(bf16), b4[None, :].astype(bf16))
    return (x[None], q_valid_mask[None])
